# baseline (speedup 1.0000x reference)
import jax
import jax.numpy as jnp
from jax.experimental import pallas as pl
from jax.experimental.pallas import tpu as pltpu

_HS = 12
_TK = _HS * 108
_NCHUNK = 27
_B = 1024


def _slab(t):
    t = jnp.minimum(t, _NCHUNK - 1)
    return t // 9, (t % 9) * _HS


def _net_kernel(x_hbm, w1_ref, b1_ref, w2_ref, b2_ref, w3_ref, b3_ref,
                w4_ref, b4_ref, w5_ref, b5_ref, o_ref, xbuf, sem, acc, xrbuf):
    k = pl.program_id(0)

    def x_copy(t, slot):
        c, h0 = _slab(t)
        return pltpu.make_async_copy(
            x_hbm.at[:, c, pl.ds(h0, _HS), :], xbuf.at[slot], sem.at[slot])

    def relayout(t):
        xb = xbuf[t % 2].astype(jnp.bfloat16)
        xrbuf[t % 2] = xb.reshape(_B, _TK)

    @pl.when(k == 0)
    def _():
        acc[...] = jnp.zeros_like(acc)
        x_copy(0, 0).start()
        x_copy(0, 0).wait()
        relayout(0)
        x_copy(1, 1).start()

    @pl.when(k + 2 < _NCHUNK)
    def _():
        x_copy(k + 2, k % 2).start()

    acc[...] += jnp.dot(
        xrbuf[k % 2], w1_ref[...].astype(jnp.bfloat16),
        preferred_element_type=jnp.float32,
    )

    @pl.when(k + 1 < _NCHUNK)
    def _():
        x_copy(k + 1, (k + 1) % 2).wait()
        relayout(k + 1)

    @pl.when(k == _NCHUNK - 1)
    def _():
        h = jnp.maximum(acc[...] + b1_ref[...], 0.0)
        for w_ref, b_ref, relu in ((w2_ref, b2_ref, True),
                                   (w3_ref, b3_ref, True),
                                   (w4_ref, b4_ref, True),
                                   (w5_ref, b5_ref, False)):
            h = jnp.dot(h.astype(jnp.bfloat16), w_ref[...].astype(jnp.bfloat16),
                        preferred_element_type=jnp.float32)
            h = h + b_ref[...]
            if relu:
                h = jnp.maximum(h, 0.0)
        o_ref[...] = h


def kernel(x, w1, b1, w2, b2, w3, b3, w4, b4, w5, b5):
    B = x.shape[0]
    N1 = w1.shape[1]

    def _const(k):
        return (0, 0)

    return pl.pallas_call(
        _net_kernel,
        out_shape=jax.ShapeDtypeStruct((B, w5.shape[1]), jnp.float32),
        grid=(_NCHUNK,),
        in_specs=[
            pl.BlockSpec(memory_space=pl.ANY),
            pl.BlockSpec((_TK, N1), lambda k: (k, 0)),
            pl.BlockSpec(b1.shape, _const),
            pl.BlockSpec(w2.shape, _const),
            pl.BlockSpec(b2.shape, _const),
            pl.BlockSpec(w3.shape, _const),
            pl.BlockSpec(b3.shape, _const),
            pl.BlockSpec(w4.shape, _const),
            pl.BlockSpec(b4.shape, _const),
            pl.BlockSpec(w5.shape, _const),
            pl.BlockSpec(b5.shape, _const),
        ],
        out_specs=pl.BlockSpec((B, w5.shape[1]), _const),
        scratch_shapes=[
            pltpu.VMEM((2, B, _HS, 108), jnp.float32),
            pltpu.SemaphoreType.DMA((2,)),
            pltpu.VMEM((B, N1), jnp.float32),
            pltpu.VMEM((2, B, _TK), jnp.bfloat16),
        ],
        compiler_params=pltpu.CompilerParams(
            dimension_semantics=("arbitrary",),
            vmem_limit_bytes=62 << 20,
        ),
    )(x, w1, b1, w2, b2, w3, b3, w4, b4, w5, b5)


# speedup vs baseline: 1.0690x; 1.0690x over previous
import jax
import jax.numpy as jnp
from jax.experimental import pallas as pl
from jax.experimental.pallas import tpu as pltpu

_HS = 12
_TK = _HS * 108
_NCHUNK = 27
_B = 1024


def _slab(t):
    t = jnp.minimum(t, _NCHUNK - 1)
    return t // 9, (t % 9) * _HS


def _net_kernel(x_hbm, w1_ref, b1_ref, w2_ref, b2_ref, w3_ref, b3_ref,
                w4_ref, b4_ref, w5_ref, b5_ref, o_ref, xbuf, sem, acc):
    k = pl.program_id(0)

    def x_copy(t, slot):
        c, h0 = _slab(t)
        return pltpu.make_async_copy(
            x_hbm.at[:, c, pl.ds(h0, _HS), :], xbuf.at[slot], sem.at[slot])

    @pl.when(k == 0)
    def _():
        acc[...] = jnp.zeros_like(acc)
        x_copy(0, 0).start()

    @pl.when(k + 1 < _NCHUNK)
    def _():
        x_copy(k + 1, (k + 1) % 2).start()

    slot = k % 2
    x_copy(k, slot).wait()
    xb = xbuf[slot].astype(jnp.bfloat16)
    xr = xb.reshape(_B, _TK)
    acc[...] += jnp.dot(
        xr, w1_ref[...].astype(jnp.bfloat16),
        preferred_element_type=jnp.float32,
    )

    @pl.when(k == _NCHUNK - 1)
    def _():
        h = jnp.maximum(acc[...] + b1_ref[...], 0.0)
        for w_ref, b_ref, relu in ((w2_ref, b2_ref, True),
                                   (w3_ref, b3_ref, True),
                                   (w4_ref, b4_ref, True),
                                   (w5_ref, b5_ref, False)):
            h = jnp.dot(h.astype(jnp.bfloat16), w_ref[...].astype(jnp.bfloat16),
                        preferred_element_type=jnp.float32)
            h = h + b_ref[...]
            if relu:
                h = jnp.maximum(h, 0.0)
        o_ref[...] = h


def kernel(x, w1, b1, w2, b2, w3, b3, w4, b4, w5, b5):
    B = x.shape[0]
    N1 = w1.shape[1]

    def _const(k):
        return (0, 0)

    return pl.pallas_call(
        _net_kernel,
        out_shape=jax.ShapeDtypeStruct((B, w5.shape[1]), jnp.float32),
        grid=(_NCHUNK,),
        in_specs=[
            pl.BlockSpec(memory_space=pl.ANY),
            pl.BlockSpec((_TK, N1), lambda k: (k, 0)),
            pl.BlockSpec(b1.shape, _const),
            pl.BlockSpec(w2.shape, _const),
            pl.BlockSpec(b2.shape, _const),
            pl.BlockSpec(w3.shape, _const),
            pl.BlockSpec(b3.shape, _const),
            pl.BlockSpec(w4.shape, _const),
            pl.BlockSpec(b4.shape, _const),
            pl.BlockSpec(w5.shape, _const),
            pl.BlockSpec(b5.shape, _const),
        ],
        out_specs=pl.BlockSpec((B, w5.shape[1]), _const),
        scratch_shapes=[
            pltpu.VMEM((2, B, _HS, 108), jnp.float32),
            pltpu.SemaphoreType.DMA((2,)),
            pltpu.VMEM((B, N1), jnp.float32),
        ],
        compiler_params=pltpu.CompilerParams(
            dimension_semantics=("arbitrary",),
            vmem_limit_bytes=62 << 20,
        ),
    )(x, w1, b1, w2, b2, w3, b3, w4, b4, w5, b5)
